# baseline (speedup 1.0000x reference)
import jax
import jax.numpy as jnp
from jax import lax
from jax.experimental import pallas as pl
from jax.experimental.pallas import tpu as pltpu
from jax.experimental.pallas import tpu_sc as plsc

B = 32768
BLK = 4096
NE = 4
D = 64
NIN = 17

NC = 2
NS = 16
NW = NC * NS
TOK = B // NW
LANES = 16
CH = TOK // LANES


def _first_eq_rows(rows, m):
    e = [jnp.where(r == m, 1.0, 0.0) for r in rows]
    f0 = e[0]
    f1 = e[1] * (1.0 - f0)
    f2 = e[2] * (1.0 - f0) * (1.0 - e[1])
    f3 = e[3] * (1.0 - f0) * (1.0 - e[1]) * (1.0 - e[2])
    return [f0, f1, f2, f3]


def _top2(w):
    m1 = jnp.maximum(jnp.maximum(w[0], w[1]), jnp.maximum(w[2], w[3]))
    f1 = _first_eq_rows(w, m1)
    neg = jnp.float32(-jnp.inf)
    wm = [jnp.where(f1[k] > 0.5, neg, w[k]) for k in range(NE)]
    m2 = jnp.maximum(jnp.maximum(wm[0], wm[1]), jnp.maximum(wm[2], wm[3]))
    f2 = _first_eq_rows(wm, m2)
    return [jnp.minimum(f1[k] + f2[k], 1.0) for k in range(NE)]


def _sc_router_body(xin_hbm, wr_hbm, br_hbm, w_hbm, xv, wrv, brv, wv):
    wid = lax.axis_index("s") * NC + lax.axis_index("c")
    pltpu.sync_copy(xin_hbm.at[wid], xv)
    pltpu.sync_copy(wr_hbm, wrv)
    pltpu.sync_copy(br_hbm, brv)

    def chunk(c, carry):
        sl = pl.ds(c * LANES, LANES)
        xs = [xv[k, sl] for k in range(NIN)]
        logits = []
        for e in range(NE):
            acc = brv[e, :]
            for k in range(NIN):
                acc = acc + xs[k] * wrv[e, k, :]
            logits.append(acc)
        m = jnp.maximum(jnp.maximum(logits[0], logits[1]),
                        jnp.maximum(logits[2], logits[3]))
        ex = [jnp.exp(l - m) for l in logits]
        s = ex[0] + ex[1] + ex[2] + ex[3]
        for e in range(NE):
            wv[e, sl] = ex[e] / s
        return carry

    lax.fori_loop(0, CH, chunk, 0)
    pltpu.sync_copy(wv, w_hbm.at[wid])


def _make_sc_router():
    return pl.kernel(
        _sc_router_body,
        out_type=jax.ShapeDtypeStruct((NW, NE, TOK), jnp.float32),
        mesh=plsc.VectorSubcoreMesh(core_axis_name="c", subcore_axis_name="s",
                                    num_cores=NC, num_subcores=NS),
        scratch_types=[
            pltpu.VMEM((NIN, TOK), jnp.float32),
            pltpu.VMEM((NE, NIN, LANES), jnp.float32),
            pltpu.VMEM((NE, LANES), jnp.float32),
            pltpu.VMEM((NE, TOK), jnp.float32),
        ],
    )


def _tc_body(t_ref, auxt_ref, wrt_ref, brc_ref, wblk_ref, emb_ref, mask_ref):
    t = t_ref[...]
    auxt = auxt_ref[...]

    rin = jnp.concatenate([t, auxt], axis=0)
    logits = jnp.dot(wrt_ref[...], rin,
                     preferred_element_type=jnp.float32)
    logits = logits + brc_ref[...]
    m = jnp.max(logits, axis=0, keepdims=True)
    e = jnp.exp(logits - m)
    s = jnp.sum(e, axis=0, keepdims=True)
    w = e / s

    wr = [w[k:k + 1] for k in range(NE)]
    sel = _top2(wr)
    disp = jnp.concatenate([wr[k] * sel[k] for k in range(NE)], axis=0)
    mask_ref[...] = jnp.concatenate(sel, axis=0)

    i8 = jax.lax.broadcasted_iota(jnp.int32, (8, 1), 0).astype(jnp.float32)
    i16 = jax.lax.broadcasted_iota(jnp.int32, (16, 1), 0).astype(jnp.float32)
    u = (i8 + 1.0) * t
    four = jnp.concatenate([jnp.sin(u), jnp.cos(u)], axis=0)

    grid = i16 * (1.0 / 15.0)
    us = (t - grid) * 8.0
    bsp = jnp.maximum(1.0 - jnp.abs(us), 0.0)
    spl = bsp * bsp * bsp

    dg = t - grid
    rk = jnp.exp(-10.0 * dg * dg)

    quo = jnp.floor(i16 * 0.25)
    tr_col = (i16 - 4.0 * quo) * (1.0 / 3.0)
    inv_sc = jnp.exp2(1.0 - quo)
    uw = (t - tr_col) * inv_sc
    uw2 = uw * uw
    wav = (1.0 - uw2) * jnp.exp(-0.5 * uw2)

    sb = jnp.concatenate([four * disp[0:1], spl * disp[1:2],
                          rk * disp[2:3], wav * disp[3:4]], axis=0)
    emb_ref[...] = jax.lax.dot_general(
        sb, wblk_ref[...], (((0,), (0,)), ((), ())),
        preferred_element_type=jnp.float32)


def kernel(timestamp_input, auxiliary_features, Wr, br,
           W_fourier, W_spline, W_rkhs, W_wavelet):
    xin = jnp.concatenate([timestamp_input, auxiliary_features], axis=1)
    xin3 = xin.reshape(NW, TOK, NIN).transpose(0, 2, 1)
    wrb = jnp.broadcast_to(Wr.T[:, :, None], (NE, NIN, LANES))
    brb = jnp.broadcast_to(br[:, None], (NE, LANES))
    w3 = _make_sc_router()(xin3, wrb, brb)
    w = w3.transpose(0, 2, 1).reshape(B, NE)

    t_row = timestamp_input.reshape(1, B)
    auxt = auxiliary_features.T
    wrt = Wr.T
    brc = br.reshape(NE, 1)
    wblk = jnp.zeros((4 * 16, 4 * D), dtype=jnp.float32)
    for i, We in enumerate((W_fourier, W_spline, W_rkhs, W_wavelet)):
        wblk = wblk.at[16 * i:16 * (i + 1), D * i:D * (i + 1)].set(We)

    emb, mask_t = pl.pallas_call(
        _tc_body,
        grid=(B // BLK,),
        in_specs=[
            pl.BlockSpec((1, BLK), lambda i: (0, i)),
            pl.BlockSpec((16, BLK), lambda i: (0, i)),
            pl.BlockSpec((NE, NIN), lambda i: (0, 0)),
            pl.BlockSpec((NE, 1), lambda i: (0, 0)),
            pl.BlockSpec((64, 4 * D), lambda i: (0, 0)),
        ],
        out_specs=[
            pl.BlockSpec((BLK, 4 * D), lambda i: (i, 0)),
            pl.BlockSpec((NE, BLK), lambda i: (0, i)),
        ],
        out_shape=[
            jax.ShapeDtypeStruct((B, 4 * D), jnp.float32),
            jax.ShapeDtypeStruct((NE, B), jnp.float32),
        ],
    )(t_row, auxt, wrt, brc, wblk)
    mask = mask_t.T.astype(jnp.bool_)

    return emb, w, mask
